# baseline (speedup 1.0000x reference)
import functools

import jax
import jax.numpy as jnp
from jax import lax
from jax.experimental import pallas as pl
from jax.experimental.pallas import tpu as pltpu
from jax.experimental.pallas import tpu_sc as plsc

V = 1000000
P = 10000
D = 16
B = 1024
CR = 1.0

_NC = 2
_NS = 16
_NW = _NC * _NS
_BPW = B // _NW
_TAIL = (V // 128) * 128
_LASTBLK = _TAIL - 128
_NBUF = 8

_ROWS = 256
_GRID = B // _ROWS


def _sc_children_body(idx_hbm, tabT_hbm, pids_hbm, resp_hbm,
                      out_hbm, femb_out, prow_out,
                      idx_v, tail_v, blkbuf, out_blk, out_blkT,
                      pids_v, qrow_v, prows_v, out_pblk, sems, sem_p, sem_r):
    wid = lax.axis_index("s") * _NC + lax.axis_index("c")
    base = wid * _BPW
    pltpu.sync_copy(idx_hbm.at[pl.ds(base, _BPW)], idx_v)
    cp_p = pltpu.async_copy(pids_hbm.at[idx_v], pids_v, sem_p)
    pltpu.sync_copy(tabT_hbm.at[:, pl.ds(_TAIL, V - _TAIL)], tail_v)
    iota = lax.iota(jnp.int32, 16)
    chunks = [idx_v[pl.ds(0, 16)], idx_v[pl.ds(16, 16)]]

    def ridx(i):
        return jnp.sum(jnp.where(iota == (i % 16), chunks[i // 16], 0))

    rs = [ridx(i) for i in range(_BPW)]
    rblks = [jnp.minimum((r // 128) * 128, _LASTBLK) for r in rs]

    def fire(i):
        s = i % _NBUF
        rblk = pl.multiple_of(rblks[i], 128)
        return pltpu.async_copy(
            tabT_hbm.at[:, pl.ds(rblk, 128)], blkbuf.at[s], sems[s])

    def extract(i):
        s = i % _NBUF
        r = rs[i]
        rblk = rblks[i]
        rmod = jnp.full((16,), (r - rblk) & 127, jnp.int32)
        rtail = jnp.full((16,), jnp.clip(r - _TAIL, 0, V - _TAIL - 1), jnp.int32)
        coli = jnp.full((16,), i, jnp.int32)
        lo_n = plsc.load_gather(blkbuf.at[s], [iota, rmod])
        hi_n = plsc.load_gather(blkbuf.at[s], [iota + 16, rmod])
        lo_t = plsc.load_gather(tail_v, [iota, rtail])
        hi_t = plsc.load_gather(tail_v, [iota + 16, rtail])
        sel = r < _TAIL
        lo = jnp.where(sel, lo_n, lo_t)
        hi = jnp.where(sel, hi_n, hi_t)
        plsc.store_scatter(out_blk, [iota, coli], lo)
        plsc.store_scatter(out_blk, [iota + 16, coli], hi)
        plsc.store_scatter(out_blkT, [coli, iota], lo)
        plsc.store_scatter(out_blkT, [coli, iota + 16], hi)

    handles = {}
    for i in range(_NBUF):
        handles[i] = fire(i)
    cp_p.wait()
    pchunks = [pids_v[pl.ds(0, 16)], pids_v[pl.ds(16, 16)]]
    qrow_v[pl.ds(0, 16)] = pchunks[0] // 4
    qrow_v[pl.ds(16, 16)] = pchunks[1] // 4
    cp_r = pltpu.async_copy(resp_hbm.at[qrow_v], prows_v, sem_r)
    def prow_extract():
        qoffs = [(pchunks[0] & 3) * 32, (pchunks[1] & 3) * 32]
        for h in range(2):
            b16 = iota + h * 16
            for c in range(2 * D):
                vals = plsc.load_gather(prows_v, [b16, qoffs[h] + c])
                plsc.store_scatter(
                    out_pblk, [b16, jnp.full((16,), c, jnp.int32)], vals)
        pltpu.sync_copy(out_pblk, prow_out.at[pl.ds(base, _BPW)])

    for i in range(_BPW):
        handles[i].wait()
        extract(i)
        if i + _NBUF < _BPW:
            handles[i + _NBUF] = fire(i + _NBUF)
        if i == 2:
            cp_r.wait()
            prow_extract()
    pltpu.sync_copy(out_blk, out_hbm.at[wid])
    pltpu.sync_copy(out_blkT, femb_out.at[pl.ds(base, _BPW)])


@functools.cache
def _sc_children():
    return pl.kernel(
        _sc_children_body,
        out_type=(jax.ShapeDtypeStruct((_NW, 2 * D, _BPW), jnp.float32),
                  jax.ShapeDtypeStruct((B, 2 * D), jnp.float32),
                  jax.ShapeDtypeStruct((B, 2 * D), jnp.float32)),
        mesh=plsc.VectorSubcoreMesh(core_axis_name="c", subcore_axis_name="s"),
        scratch_types=[
            pltpu.VMEM((_BPW,), jnp.int32),
            pltpu.VMEM((2 * D, V - _TAIL), jnp.float32),
            pltpu.VMEM((_NBUF, 2 * D, 128), jnp.float32),
            pltpu.VMEM((2 * D, _BPW), jnp.float32),
            pltpu.VMEM((_BPW, 2 * D), jnp.float32),
            pltpu.VMEM((_BPW,), jnp.int32),
            pltpu.VMEM((_BPW,), jnp.int32),
            pltpu.VMEM((_BPW, 128), jnp.float32),
            pltpu.VMEM((_BPW, 2 * D), jnp.float32),
            [pltpu.SemaphoreType.DMA] * _NBUF,
            pltpu.SemaphoreType.DMA,
            pltpu.SemaphoreType.DMA,
        ],
        compiler_params=pltpu.CompilerParams(
            use_tc_tiling_on_sc=True, needs_layout_passes=False),
    )


def _tc_dist_body(femb_ref, out3_ref, prow_ref, loss_ref, lower_ref, higher_ref,
                  fembT_ref):
    i = pl.program_id(0)

    @pl.when(i == 0)
    def _():
        for w in range(_NW):
            fembT_ref[:, pl.ds(w * _BPW, _BPW)] = out3_ref[w]
    cL = femb_ref[:, :D]
    cH = femb_ref[:, D:]
    accL = jnp.zeros((_ROWS, B), jnp.float32)
    accH = jnp.zeros((_ROWS, B), jnp.float32)
    for d in range(D):
        accL = accL + jnp.abs(cL[:, d:d + 1] - fembT_ref[d:d + 1, :])
        accH = accH + jnp.abs(cH[:, d:d + 1] - fembT_ref[D + d:D + d + 1, :])
    lower_ref[...] = accL
    higher_ref[...] = accH

    pL = prow_ref[:, :D] + CR
    pH = prow_ref[:, D:] + CR
    part = (jnp.sum(jnp.maximum(pL - cL, 0.0))
            + jnp.sum(jnp.maximum(cH - pH, 0.0))
            + jnp.sum(jnp.maximum(pL - cH, 0.0))
            + jnp.sum(jnp.maximum(cL - pH, 0.0)))

    @pl.when(i == 0)
    def _():
        loss_ref[0, 0] = 0.0

    loss_ref[0, 0] += part


_tc_dist = pl.pallas_call(
    _tc_dist_body,
    grid=(_GRID,),
    in_specs=[
        pl.BlockSpec((_ROWS, 2 * D), lambda i: (i, 0)),
        pl.BlockSpec((_NW, 2 * D, _BPW), lambda i: (0, 0, 0)),
        pl.BlockSpec((_ROWS, 2 * D), lambda i: (i, 0)),
    ],
    scratch_shapes=[pltpu.VMEM((2 * D, B), jnp.float32)],
    out_specs=[
        pl.BlockSpec(memory_space=pltpu.SMEM),
        pl.BlockSpec((_ROWS, B), lambda i: (i, 0)),
        pl.BlockSpec((_ROWS, B), lambda i: (i, 0)),
    ],
    out_shape=[
        jax.ShapeDtypeStruct((1, 1), jnp.float32),
        jax.ShapeDtypeStruct((B, B), jnp.float32),
        jax.ShapeDtypeStruct((B, B), jnp.float32),
    ],
)


@jax.jit
def kernel(idIndexes, omegaEmb, epoch, childrenEmbedding, res, parentIds):
    idx = idIndexes.astype(jnp.int32)
    ptab = parentIds.astype(jnp.int32)
    resp = res.reshape(P * D // 64, 128)
    out3, femb, prow = _sc_children()(idx, childrenEmbedding.T, ptab, resp)
    loss, lower, higher = _tc_dist(femb, out3, prow)
    return (loss[0, 0], lower, higher)


# speedup vs baseline: 1.0285x; 1.0285x over previous
import functools

import jax
import jax.numpy as jnp
from jax import lax
from jax.experimental import pallas as pl
from jax.experimental.pallas import tpu as pltpu
from jax.experimental.pallas import tpu_sc as plsc

V = 1000000
P = 10000
D = 16
B = 1024
CR = 1.0

_NC = 2
_NS = 16
_NW = _NC * _NS
_BPW = B // _NW
_TAIL = (V // 128) * 128
_LASTBLK = _TAIL - 128
_NBUF = 8

_ROWS = 256
_GRID = B // _ROWS


def _sc_children_body(idx_hbm, tabT_hbm, pids_hbm, resp_hbm,
                      out_hbm, femb_out, prow_out,
                      idx_v, tail_v, blkbuf, out_blk, out_blkT,
                      pids_v, prows_v, sems, sem_p, sem_r):
    wid = lax.axis_index("s") * _NC + lax.axis_index("c")
    base = wid * _BPW
    pltpu.sync_copy(idx_hbm.at[pl.ds(base, _BPW)], idx_v)
    cp_p = pltpu.async_copy(pids_hbm.at[idx_v], pids_v, sem_p)
    pltpu.sync_copy(tabT_hbm.at[:, pl.ds(_TAIL, V - _TAIL)], tail_v)
    iota = lax.iota(jnp.int32, 16)
    chunks = [idx_v[pl.ds(0, 16)], idx_v[pl.ds(16, 16)]]

    def ridx(i):
        return jnp.sum(jnp.where(iota == (i % 16), chunks[i // 16], 0))

    rs = [ridx(i) for i in range(_BPW)]
    rblks = [jnp.minimum((r // 128) * 128, _LASTBLK) for r in rs]

    def fire(i):
        s = i % _NBUF
        rblk = pl.multiple_of(rblks[i], 128)
        return pltpu.async_copy(
            tabT_hbm.at[:, pl.ds(rblk, 128)], blkbuf.at[s], sems[s])

    def extract(i):
        s = i % _NBUF
        r = rs[i]
        rblk = rblks[i]
        rmod = jnp.full((16,), (r - rblk) & 127, jnp.int32)
        rtail = jnp.full((16,), jnp.clip(r - _TAIL, 0, V - _TAIL - 1), jnp.int32)
        coli = jnp.full((16,), i, jnp.int32)
        lo_n = plsc.load_gather(blkbuf.at[s], [iota, rmod])
        hi_n = plsc.load_gather(blkbuf.at[s], [iota + 16, rmod])
        lo_t = plsc.load_gather(tail_v, [iota, rtail])
        hi_t = plsc.load_gather(tail_v, [iota + 16, rtail])
        sel = r < _TAIL
        lo = jnp.where(sel, lo_n, lo_t)
        hi = jnp.where(sel, hi_n, hi_t)
        plsc.store_scatter(out_blk, [iota, coli], lo)
        plsc.store_scatter(out_blk, [iota + 16, coli], hi)
        plsc.store_scatter(out_blkT, [coli, iota], lo)
        plsc.store_scatter(out_blkT, [coli, iota + 16], hi)

    handles = {}
    for i in range(_NBUF):
        handles[i] = fire(i)
    cp_p.wait()
    cp_r = pltpu.async_copy(resp_hbm.at[pids_v], prows_v, sem_r)
    for i in range(_BPW):
        handles[i].wait()
        extract(i)
        if i + _NBUF < _BPW:
            handles[i + _NBUF] = fire(i + _NBUF)
    pltpu.sync_copy(out_blk, out_hbm.at[wid])
    pltpu.sync_copy(out_blkT, femb_out.at[pl.ds(base, _BPW)])
    cp_r.wait()
    pltpu.sync_copy(prows_v, prow_out.at[pl.ds(base, _BPW)])


@functools.cache
def _sc_children():
    return pl.kernel(
        _sc_children_body,
        out_type=(jax.ShapeDtypeStruct((_NW, 2 * D, _BPW), jnp.float32),
                  jax.ShapeDtypeStruct((B, 2 * D), jnp.float32),
                  jax.ShapeDtypeStruct((B, 128), jnp.float32)),
        mesh=plsc.VectorSubcoreMesh(core_axis_name="c", subcore_axis_name="s"),
        scratch_types=[
            pltpu.VMEM((_BPW,), jnp.int32),
            pltpu.VMEM((2 * D, V - _TAIL), jnp.float32),
            pltpu.VMEM((_NBUF, 2 * D, 128), jnp.float32),
            pltpu.VMEM((2 * D, _BPW), jnp.float32),
            pltpu.VMEM((_BPW, 2 * D), jnp.float32),
            pltpu.VMEM((_BPW,), jnp.int32),
            pltpu.VMEM((_BPW, 128), jnp.float32),
            [pltpu.SemaphoreType.DMA] * _NBUF,
            pltpu.SemaphoreType.DMA,
            pltpu.SemaphoreType.DMA,
        ],
        compiler_params=pltpu.CompilerParams(
            use_tc_tiling_on_sc=True, needs_layout_passes=False),
    )


def _tc_dist_body(femb_ref, out3_ref, prow_ref, loss_ref, lower_ref, higher_ref,
                  fembT_ref):
    i = pl.program_id(0)

    @pl.when(i == 0)
    def _():
        for w in range(_NW):
            fembT_ref[:, pl.ds(w * _BPW, _BPW)] = out3_ref[w]
    cL = femb_ref[:, :D]
    cH = femb_ref[:, D:]
    accL = jnp.zeros((_ROWS, B), jnp.float32)
    accH = jnp.zeros((_ROWS, B), jnp.float32)
    for d in range(D):
        accL = accL + jnp.abs(cL[:, d:d + 1] - fembT_ref[d:d + 1, :])
        accH = accH + jnp.abs(cH[:, d:d + 1] - fembT_ref[D + d:D + d + 1, :])
    lower_ref[...] = accL
    higher_ref[...] = accH

    pL = prow_ref[:, :D] + CR
    pH = prow_ref[:, D:2 * D] + CR
    part = (jnp.sum(jnp.maximum(pL - cL, 0.0))
            + jnp.sum(jnp.maximum(cH - pH, 0.0))
            + jnp.sum(jnp.maximum(pL - cH, 0.0))
            + jnp.sum(jnp.maximum(cL - pH, 0.0)))

    @pl.when(i == 0)
    def _():
        loss_ref[0, 0] = 0.0

    loss_ref[0, 0] += part


_tc_dist = pl.pallas_call(
    _tc_dist_body,
    grid=(_GRID,),
    in_specs=[
        pl.BlockSpec((_ROWS, 2 * D), lambda i: (i, 0)),
        pl.BlockSpec((_NW, 2 * D, _BPW), lambda i: (0, 0, 0)),
        pl.BlockSpec((_ROWS, 128), lambda i: (i, 0)),
    ],
    scratch_shapes=[pltpu.VMEM((2 * D, B), jnp.float32)],
    out_specs=[
        pl.BlockSpec(memory_space=pltpu.SMEM),
        pl.BlockSpec((_ROWS, B), lambda i: (i, 0)),
        pl.BlockSpec((_ROWS, B), lambda i: (i, 0)),
    ],
    out_shape=[
        jax.ShapeDtypeStruct((1, 1), jnp.float32),
        jax.ShapeDtypeStruct((B, B), jnp.float32),
        jax.ShapeDtypeStruct((B, B), jnp.float32),
    ],
)


@jax.jit
def kernel(idIndexes, omegaEmb, epoch, childrenEmbedding, res, parentIds):
    idx = idIndexes.astype(jnp.int32)
    ptab = parentIds.astype(jnp.int32)
    resp = jnp.pad(res, ((0, 0), (0, 128 - 2 * D)))
    out3, femb, prow = _sc_children()(idx, childrenEmbedding.T, ptab, resp)
    loss, lower, higher = _tc_dist(femb, out3, prow)
    return (loss[0, 0], lower, higher)


# speedup vs baseline: 1.0298x; 1.0013x over previous
import functools

import jax
import jax.numpy as jnp
from jax import lax
from jax.experimental import pallas as pl
from jax.experimental.pallas import tpu as pltpu
from jax.experimental.pallas import tpu_sc as plsc

V = 1000000
P = 10000
D = 16
B = 1024
CR = 1.0

_NC = 2
_NS = 16
_NW = _NC * _NS
_BPW = B // _NW
_TAIL = (V // 128) * 128
_LASTBLK = _TAIL - 128
_NBUF = 8

_ROWS = 256
_GRID = B // _ROWS


def _sc_children_body(idx_hbm, tabT_hbm, pids_hbm, resp_hbm,
                      out_hbm, femb_out, prow_out,
                      idx_v, tail_v, blkbuf, out_blk, out_blkT,
                      pids_v, prows_v, sems, sem_p, sem_r):
    wid = lax.axis_index("s") * _NC + lax.axis_index("c")
    base = wid * _BPW
    pltpu.sync_copy(idx_hbm.at[pl.ds(base, _BPW)], idx_v)
    cp_p = pltpu.async_copy(pids_hbm.at[idx_v], pids_v, sem_p)
    pltpu.sync_copy(tabT_hbm.at[:, pl.ds(_TAIL, V - _TAIL)], tail_v)
    iota = lax.iota(jnp.int32, 16)
    chunks = [idx_v[pl.ds(0, 16)], idx_v[pl.ds(16, 16)]]

    def ridx(i):
        return jnp.sum(jnp.where(iota == (i % 16), chunks[i // 16], 0))

    rs = [ridx(i) for i in range(_BPW)]
    rblks = [jnp.minimum((r // 128) * 128, _LASTBLK) for r in rs]

    def fire(i):
        s = i % _NBUF
        rblk = pl.multiple_of(rblks[i], 128)
        return pltpu.async_copy(
            tabT_hbm.at[:, pl.ds(rblk, 128)], blkbuf.at[s], sems[s])

    def extract(i):
        s = i % _NBUF
        r = rs[i]
        rblk = rblks[i]
        rmod = jnp.full((16,), (r - rblk) & 127, jnp.int32)
        rtail = jnp.full((16,), jnp.clip(r - _TAIL, 0, V - _TAIL - 1), jnp.int32)
        coli = jnp.full((16,), i, jnp.int32)
        lo_n = plsc.load_gather(blkbuf.at[s], [iota, rmod])
        hi_n = plsc.load_gather(blkbuf.at[s], [iota + 16, rmod])
        lo_t = plsc.load_gather(tail_v, [iota, rtail])
        hi_t = plsc.load_gather(tail_v, [iota + 16, rtail])
        sel = r < _TAIL
        lo = jnp.where(sel, lo_n, lo_t)
        hi = jnp.where(sel, hi_n, hi_t)
        plsc.store_scatter(out_blk, [iota, coli], lo)
        plsc.store_scatter(out_blk, [iota + 16, coli], hi)
        plsc.store_scatter(out_blkT, [coli, iota], lo)
        plsc.store_scatter(out_blkT, [coli, iota + 16], hi)

    handles = {}
    for i in range(_NBUF):
        handles[i] = fire(i)
    cp_p.wait()
    cp_r = pltpu.async_copy(resp_hbm.at[pids_v], prows_v, sem_r)
    for i in range(_BPW):
        handles[i].wait()
        extract(i)
        if i + _NBUF < _BPW:
            handles[i + _NBUF] = fire(i + _NBUF)
    pltpu.sync_copy(out_blk, out_hbm.at[wid])
    pltpu.sync_copy(out_blkT, femb_out.at[pl.ds(base, _BPW)])
    cp_r.wait()
    pltpu.sync_copy(prows_v, prow_out.at[pl.ds(base, _BPW)])


@functools.cache
def _sc_children():
    return pl.kernel(
        _sc_children_body,
        out_type=(jax.ShapeDtypeStruct((_NW, 2 * D, _BPW), jnp.float32),
                  jax.ShapeDtypeStruct((B, 2 * D), jnp.float32),
                  jax.ShapeDtypeStruct((B, 128), jnp.float32)),
        mesh=plsc.VectorSubcoreMesh(core_axis_name="c", subcore_axis_name="s"),
        scratch_types=[
            pltpu.VMEM((_BPW,), jnp.int32),
            pltpu.VMEM((2 * D, V - _TAIL), jnp.float32),
            pltpu.VMEM((_NBUF, 2 * D, 128), jnp.float32),
            pltpu.VMEM((2 * D, _BPW), jnp.float32),
            pltpu.VMEM((_BPW, 2 * D), jnp.float32),
            pltpu.VMEM((_BPW,), jnp.int32),
            pltpu.VMEM((_BPW, 128), jnp.float32),
            [pltpu.SemaphoreType.DMA] * _NBUF,
            pltpu.SemaphoreType.DMA,
            pltpu.SemaphoreType.DMA,
        ],
        compiler_params=pltpu.CompilerParams(
            use_tc_tiling_on_sc=True, needs_layout_passes=False,
            skip_device_barrier=True),
    )


def _tc_dist_body(femb_ref, out3_ref, prow_ref, loss_ref, lower_ref, higher_ref,
                  fembT_ref):
    i = pl.program_id(0)

    @pl.when(i == 0)
    def _():
        for w in range(_NW):
            fembT_ref[:, pl.ds(w * _BPW, _BPW)] = out3_ref[w]
    cL = femb_ref[:, :D]
    cH = femb_ref[:, D:]
    accL = jnp.zeros((_ROWS, B), jnp.float32)
    accH = jnp.zeros((_ROWS, B), jnp.float32)
    for d in range(D):
        accL = accL + jnp.abs(cL[:, d:d + 1] - fembT_ref[d:d + 1, :])
        accH = accH + jnp.abs(cH[:, d:d + 1] - fembT_ref[D + d:D + d + 1, :])
    lower_ref[...] = accL
    higher_ref[...] = accH

    pL = prow_ref[:, :D] + CR
    pH = prow_ref[:, D:2 * D] + CR
    part = (jnp.sum(jnp.maximum(pL - cL, 0.0))
            + jnp.sum(jnp.maximum(cH - pH, 0.0))
            + jnp.sum(jnp.maximum(pL - cH, 0.0))
            + jnp.sum(jnp.maximum(cL - pH, 0.0)))

    @pl.when(i == 0)
    def _():
        loss_ref[0, 0] = 0.0

    loss_ref[0, 0] += part


_tc_dist = pl.pallas_call(
    _tc_dist_body,
    grid=(_GRID,),
    in_specs=[
        pl.BlockSpec((_ROWS, 2 * D), lambda i: (i, 0)),
        pl.BlockSpec((_NW, 2 * D, _BPW), lambda i: (0, 0, 0)),
        pl.BlockSpec((_ROWS, 128), lambda i: (i, 0)),
    ],
    scratch_shapes=[pltpu.VMEM((2 * D, B), jnp.float32)],
    out_specs=[
        pl.BlockSpec(memory_space=pltpu.SMEM),
        pl.BlockSpec((_ROWS, B), lambda i: (i, 0)),
        pl.BlockSpec((_ROWS, B), lambda i: (i, 0)),
    ],
    out_shape=[
        jax.ShapeDtypeStruct((1, 1), jnp.float32),
        jax.ShapeDtypeStruct((B, B), jnp.float32),
        jax.ShapeDtypeStruct((B, B), jnp.float32),
    ],
)


@jax.jit
def kernel(idIndexes, omegaEmb, epoch, childrenEmbedding, res, parentIds):
    idx = idIndexes.astype(jnp.int32)
    ptab = parentIds.astype(jnp.int32)
    resp = jnp.pad(res, ((0, 0), (0, 128 - 2 * D)))
    out3, femb, prow = _sc_children()(idx, childrenEmbedding.T, ptab, resp)
    loss, lower, higher = _tc_dist(femb, out3, prow)
    return (loss[0, 0], lower, higher)


# speedup vs baseline: 1.0339x; 1.0040x over previous
import functools

import jax
import jax.numpy as jnp
from jax import lax
from jax.experimental import pallas as pl
from jax.experimental.pallas import tpu as pltpu
from jax.experimental.pallas import tpu_sc as plsc

V = 1000000
P = 10000
D = 16
B = 1024
CR = 1.0

_NC = 2
_NS = 16
_NW = _NC * _NS
_BPW = B // _NW
_TAIL = (V // 128) * 128
_LASTBLK = _TAIL - 128
_NBUF = 8

_ROWS = 512
_GRID = B // _ROWS


def _sc_children_body(idx_hbm, tabT_hbm, pids_hbm, resp_hbm,
                      out_hbm, femb_out, prow_out,
                      idx_v, tail_v, blkbuf, out_blk, out_blkT,
                      pids_v, prows_v, sems, sem_p, sem_r):
    wid = lax.axis_index("s") * _NC + lax.axis_index("c")
    base = wid * _BPW
    pltpu.sync_copy(idx_hbm.at[pl.ds(base, _BPW)], idx_v)
    cp_p = pltpu.async_copy(pids_hbm.at[idx_v], pids_v, sem_p)
    pltpu.sync_copy(tabT_hbm.at[:, pl.ds(_TAIL, V - _TAIL)], tail_v)
    iota = lax.iota(jnp.int32, 16)
    chunks = [idx_v[pl.ds(0, 16)], idx_v[pl.ds(16, 16)]]

    def ridx(i):
        return jnp.sum(jnp.where(iota == (i % 16), chunks[i // 16], 0))

    rs = [ridx(i) for i in range(_BPW)]
    rblks = [jnp.minimum((r // 128) * 128, _LASTBLK) for r in rs]

    def fire(i):
        s = i % _NBUF
        rblk = pl.multiple_of(rblks[i], 128)
        return pltpu.async_copy(
            tabT_hbm.at[:, pl.ds(rblk, 128)], blkbuf.at[s], sems[s])

    def extract(i):
        s = i % _NBUF
        r = rs[i]
        rblk = rblks[i]
        rmod = jnp.full((16,), (r - rblk) & 127, jnp.int32)
        rtail = jnp.full((16,), jnp.clip(r - _TAIL, 0, V - _TAIL - 1), jnp.int32)
        coli = jnp.full((16,), i, jnp.int32)
        lo_n = plsc.load_gather(blkbuf.at[s], [iota, rmod])
        hi_n = plsc.load_gather(blkbuf.at[s], [iota + 16, rmod])
        lo_t = plsc.load_gather(tail_v, [iota, rtail])
        hi_t = plsc.load_gather(tail_v, [iota + 16, rtail])
        sel = r < _TAIL
        lo = jnp.where(sel, lo_n, lo_t)
        hi = jnp.where(sel, hi_n, hi_t)
        plsc.store_scatter(out_blk, [iota, coli], lo)
        plsc.store_scatter(out_blk, [iota + 16, coli], hi)
        plsc.store_scatter(out_blkT, [coli, iota], lo)
        plsc.store_scatter(out_blkT, [coli, iota + 16], hi)

    handles = {}
    for i in range(_NBUF):
        handles[i] = fire(i)
    cp_p.wait()
    cp_r = pltpu.async_copy(resp_hbm.at[pids_v], prows_v, sem_r)
    for i in range(_BPW):
        handles[i].wait()
        extract(i)
        if i + _NBUF < _BPW:
            handles[i + _NBUF] = fire(i + _NBUF)
    pltpu.sync_copy(out_blk, out_hbm.at[wid])
    pltpu.sync_copy(out_blkT, femb_out.at[pl.ds(base, _BPW)])
    cp_r.wait()
    pltpu.sync_copy(prows_v, prow_out.at[pl.ds(base, _BPW)])


@functools.cache
def _sc_children():
    return pl.kernel(
        _sc_children_body,
        out_type=(jax.ShapeDtypeStruct((_NW, 2 * D, _BPW), jnp.float32),
                  jax.ShapeDtypeStruct((B, 2 * D), jnp.float32),
                  jax.ShapeDtypeStruct((B, 128), jnp.float32)),
        mesh=plsc.VectorSubcoreMesh(core_axis_name="c", subcore_axis_name="s"),
        scratch_types=[
            pltpu.VMEM((_BPW,), jnp.int32),
            pltpu.VMEM((2 * D, V - _TAIL), jnp.float32),
            pltpu.VMEM((_NBUF, 2 * D, 128), jnp.float32),
            pltpu.VMEM((2 * D, _BPW), jnp.float32),
            pltpu.VMEM((_BPW, 2 * D), jnp.float32),
            pltpu.VMEM((_BPW,), jnp.int32),
            pltpu.VMEM((_BPW, 128), jnp.float32),
            [pltpu.SemaphoreType.DMA] * _NBUF,
            pltpu.SemaphoreType.DMA,
            pltpu.SemaphoreType.DMA,
        ],
        compiler_params=pltpu.CompilerParams(
            use_tc_tiling_on_sc=True, needs_layout_passes=False,
            skip_device_barrier=True),
    )


def _tc_dist_body(femb_ref, out3_ref, prow_ref, loss_ref, lower_ref, higher_ref,
                  fembT_ref):
    i = pl.program_id(0)

    @pl.when(i == 0)
    def _():
        for w in range(_NW):
            fembT_ref[:, pl.ds(w * _BPW, _BPW)] = out3_ref[w]
    cL = femb_ref[:, :D]
    cH = femb_ref[:, D:]
    accL = jnp.zeros((_ROWS, B), jnp.float32)
    accH = jnp.zeros((_ROWS, B), jnp.float32)
    for d in range(D):
        accL = accL + jnp.abs(cL[:, d:d + 1] - fembT_ref[d:d + 1, :])
        accH = accH + jnp.abs(cH[:, d:d + 1] - fembT_ref[D + d:D + d + 1, :])
    lower_ref[...] = accL
    higher_ref[...] = accH

    pL = prow_ref[:, :D] + CR
    pH = prow_ref[:, D:2 * D] + CR
    part = (jnp.sum(jnp.maximum(pL - cL, 0.0))
            + jnp.sum(jnp.maximum(cH - pH, 0.0))
            + jnp.sum(jnp.maximum(pL - cH, 0.0))
            + jnp.sum(jnp.maximum(cL - pH, 0.0)))

    @pl.when(i == 0)
    def _():
        loss_ref[0, 0] = 0.0

    loss_ref[0, 0] += part


_tc_dist = pl.pallas_call(
    _tc_dist_body,
    grid=(_GRID,),
    in_specs=[
        pl.BlockSpec((_ROWS, 2 * D), lambda i: (i, 0)),
        pl.BlockSpec((_NW, 2 * D, _BPW), lambda i: (0, 0, 0)),
        pl.BlockSpec((_ROWS, 128), lambda i: (i, 0)),
    ],
    scratch_shapes=[pltpu.VMEM((2 * D, B), jnp.float32)],
    out_specs=[
        pl.BlockSpec(memory_space=pltpu.SMEM),
        pl.BlockSpec((_ROWS, B), lambda i: (i, 0)),
        pl.BlockSpec((_ROWS, B), lambda i: (i, 0)),
    ],
    out_shape=[
        jax.ShapeDtypeStruct((1, 1), jnp.float32),
        jax.ShapeDtypeStruct((B, B), jnp.float32),
        jax.ShapeDtypeStruct((B, B), jnp.float32),
    ],
)


@jax.jit
def kernel(idIndexes, omegaEmb, epoch, childrenEmbedding, res, parentIds):
    idx = idIndexes.astype(jnp.int32)
    ptab = parentIds.astype(jnp.int32)
    resp = jnp.pad(res, ((0, 0), (0, 128 - 2 * D)))
    out3, femb, prow = _sc_children()(idx, childrenEmbedding.T, ptab, resp)
    loss, lower, higher = _tc_dist(femb, out3, prow)
    return (loss[0, 0], lower, higher)
